# Initial kernel scaffold; baseline (speedup 1.0000x reference)
import jax
import jax.numpy as jnp
from jax.experimental import pallas as pl


def kernel(x, edge_index, batch_index, topk_weight, W0, b0, W1, b1, W2, b2, g0, be0, g1, be1, g2, be2, linW, linb):
    raise NotImplementedError("write your pallas kernel here")


# speedup vs baseline: 1.7499x; 1.7499x over previous
import jax
import jax.numpy as jnp
from jax.experimental import pallas as pl
from jax.experimental.pallas import tpu as pltpu

_N = 10000
_G = 64
_K_MAX = 1024
_EK_MAX = 8192
_MIN_SCORE = 0.1


def _net_kernel(scal_ref, cer_ref, cec_ref, bs_ref, xs_ref,
                w0_ref, b0_ref, g0_ref, be0_ref,
                w1_ref, b1_ref, g1_ref, be1_ref,
                w2_ref, b2_ref, g2_ref, be2_ref,
                lw_ref, lb_ref, out_ref, adj_ref):
    k = scal_ref[0]
    ek = scal_ref[1]
    kf = k.astype(jnp.float32)

    adj_ref[...] = jnp.zeros((_K_MAX, _K_MAX), jnp.float32)
    lane = jax.lax.broadcasted_iota(jnp.int32, (1, _K_MAX), 1)

    def body(i, carry):
        r = cer_ref[i]
        c = cec_ref[i]
        adj_ref[pl.ds(c, 1), :] += (lane == r).astype(jnp.float32)
        return carry

    jax.lax.fori_loop(0, ek, body, 0)
    cnt = adj_ref[...]

    riota = jax.lax.broadcasted_iota(jnp.int32, (_K_MAX, 1), 0)
    validf = (riota < k).astype(jnp.float32)
    deg = jnp.sum(cnt, axis=1, keepdims=True) + validf
    dinv = jnp.where(deg > 0, jax.lax.rsqrt(jnp.maximum(deg, 1e-12)), 0.0)

    def prop(h):
        hd = h * dinv
        out = jnp.dot(cnt, hd, preferred_element_type=jnp.float32, precision=jax.lax.Precision.HIGHEST)
        return out * dinv + (validf * dinv * dinv) * h

    def bn(h, g, be):
        m = jnp.sum(h * validf, axis=0, keepdims=True) / kf
        c = (h - m) * validf
        v = jnp.sum(c * c, axis=0, keepdims=True) / kf
        return (h - m) * jax.lax.rsqrt(v + 1e-5) * g + be

    def wdot(a, b):
        return jnp.dot(a.astype(jnp.bfloat16), b.astype(jnp.bfloat16),
                       preferred_element_type=jnp.float32)

    h = wdot(xs_ref[...], w0_ref[...])
    h = jax.nn.relu(prop(h) + b0_ref[...])
    h = bn(h, g0_ref[...], be0_ref[...])

    h = wdot(h, w1_ref[...])
    h = jax.nn.relu(prop(h) + b1_ref[...])
    h = bn(h, g1_ref[...], be1_ref[...])

    h = wdot(h, w2_ref[...])
    h = jax.nn.relu(prop(h) + b2_ref[...])
    h = bn(h, g2_ref[...], be2_ref[...])

    bsv = bs_ref[...]
    rows = []
    for g in range(_G):
        hm = jnp.where(bsv == g, h, -jnp.inf)
        rows.append(jnp.max(hm, axis=0, keepdims=True))
    pooled = jnp.concatenate(rows, axis=0)

    logits = wdot(pooled, lw_ref[...]) + lb_ref[...]
    colm = jax.lax.broadcasted_iota(jnp.int32, (1, 128), 1) < 3
    z = jnp.where(colm, logits, -jnp.inf)
    mx = jnp.max(z, axis=1, keepdims=True)
    lse = mx + jnp.log(jnp.sum(jnp.where(colm, jnp.exp(z - mx), 0.0),
                               axis=1, keepdims=True))
    out_ref[...] = z - lse


def kernel(x, edge_index, batch_index, topk_weight,
           W0, b0, W1, b1, W2, b2,
           g0, be0, g1, be1, g2, be2, linW, linb):
    raw = (x * topk_weight[None, :]).sum(axis=-1)
    smax_g = jax.ops.segment_max(raw, batch_index, num_segments=_G)
    e = jnp.exp(raw - smax_g[batch_index])
    ssum = jax.ops.segment_sum(e, batch_index, num_segments=_G)
    score = e / ssum[batch_index]
    scmax = jax.ops.segment_max(score, batch_index,
                                num_segments=_G)[batch_index] - 1e-7
    mask = score > jnp.minimum(scmax, _MIN_SCORE)
    k = jnp.sum(mask.astype(jnp.int32))
    new_idx = jnp.cumsum(mask.astype(jnp.int32)) - 1

    perm = jnp.nonzero(mask, size=_K_MAX, fill_value=_N)[0]
    sperm = jnp.take(score, perm, mode='fill', fill_value=0.0)
    xperm = jnp.take(x, perm, axis=0, mode='fill', fill_value=0.0)
    xs = xperm * sperm[:, None]
    bs = jnp.take(batch_index, perm, mode='fill',
                  fill_value=_G).astype(jnp.int32)

    row, col = edge_index[0], edge_index[1]
    evalid = mask[row] & mask[col]
    pos = jnp.cumsum(evalid.astype(jnp.int32)) - 1
    posc = jnp.where(evalid, pos, _EK_MAX)
    er = jnp.minimum(new_idx[row], _K_MAX - 1)
    ec = jnp.minimum(new_idx[col], _K_MAX - 1)
    cer = jnp.zeros((_EK_MAX,), jnp.int32).at[posc].set(er, mode='drop')
    cec = jnp.zeros((_EK_MAX,), jnp.int32).at[posc].set(ec, mode='drop')
    ek = jnp.minimum(jnp.sum(evalid.astype(jnp.int32)), _EK_MAX)

    scal = jnp.stack([k, ek]).astype(jnp.int32)

    xs8 = jnp.pad(xs, ((0, 0), (0, 6)))
    W0p = jnp.pad(W0, ((0, 6), (0, 0)))
    lWp = jnp.pad(linW, ((0, 0), (0, 125)))
    lbp = jnp.pad(linb, (0, 125)).reshape(1, 128)
    r1 = lambda a: a.reshape(1, -1)

    smem = pl.BlockSpec(memory_space=pltpu.SMEM)
    vmem = pl.BlockSpec(memory_space=pltpu.VMEM)
    out = pl.pallas_call(
        _net_kernel,
        out_shape=jax.ShapeDtypeStruct((_G, 128), jnp.float32),
        in_specs=[smem, smem, smem] + [vmem] * 16,
        out_specs=vmem,
        scratch_shapes=[pltpu.VMEM((_K_MAX, _K_MAX), jnp.float32)],
    )(scal, cer, cec, bs.reshape(_K_MAX, 1), xs8,
      W0p, r1(b0), r1(g0), r1(be0),
      W1, r1(b1), r1(g1), r1(be1),
      W2, r1(b2), r1(g2), r1(be2),
      lWp, lbp)
    return out[:, :3]


# speedup vs baseline: 1.7536x; 1.0022x over previous
import jax
import jax.numpy as jnp
from jax.experimental import pallas as pl
from jax.experimental.pallas import tpu as pltpu

_N = 10000
_G = 64
_K_MAX = 1024
_EK_MAX = 8192
_MIN_SCORE = 0.1


def _net_kernel(scal_ref, cer_ref, cec_ref, bs_ref, xs_ref,
                w0_ref, b0_ref, g0_ref, be0_ref,
                w1_ref, b1_ref, g1_ref, be1_ref,
                w2_ref, b2_ref, g2_ref, be2_ref,
                lw_ref, lb_ref, out_ref, adj_ref):
    k = scal_ref[0]
    ek = scal_ref[1]
    kf = k.astype(jnp.float32)

    adj_ref[...] = jnp.zeros((_K_MAX, _K_MAX), jnp.float32)
    lane = jax.lax.broadcasted_iota(jnp.int32, (1, _K_MAX), 1)

    def body(i, carry):
        r = cer_ref[i]
        c = cec_ref[i]
        adj_ref[pl.ds(c, 1), :] += (lane == r).astype(jnp.float32)
        return carry

    jax.lax.fori_loop(0, ek, body, 0)
    cnt = adj_ref[...]

    riota = jax.lax.broadcasted_iota(jnp.int32, (_K_MAX, 1), 0)
    validf = (riota < k).astype(jnp.float32)
    deg = jnp.sum(cnt, axis=1, keepdims=True) + validf
    dinv = jnp.where(deg > 0, jax.lax.rsqrt(jnp.maximum(deg, 1e-12)), 0.0)

    def prop(h):
        hd = h * dinv
        out = jnp.dot(cnt, hd, preferred_element_type=jnp.float32, precision=jax.lax.Precision.HIGHEST)
        return out * dinv + (validf * dinv * dinv) * h

    def bn(h, g, be):
        m = jnp.sum(h * validf, axis=0, keepdims=True) / kf
        c = (h - m) * validf
        v = jnp.sum(c * c, axis=0, keepdims=True) / kf
        return (h - m) * jax.lax.rsqrt(v + 1e-5) * g + be

    def wdot(a, b):
        return jnp.dot(a.astype(jnp.bfloat16), b.astype(jnp.bfloat16),
                       preferred_element_type=jnp.float32)

    h = wdot(xs_ref[...], w0_ref[...])
    h = jax.nn.relu(prop(h) + b0_ref[...])
    h = bn(h, g0_ref[...], be0_ref[...])

    h = wdot(h, w1_ref[...])
    h = jax.nn.relu(prop(h) + b1_ref[...])
    h = bn(h, g1_ref[...], be1_ref[...])

    h = wdot(h, w2_ref[...])
    h = jax.nn.relu(prop(h) + b2_ref[...])
    h = bn(h, g2_ref[...], be2_ref[...])

    bsv = bs_ref[...]
    rows = []
    for g in range(_G):
        hm = jnp.where(bsv == g, h, -jnp.inf)
        rows.append(jnp.max(hm, axis=0, keepdims=True))
    pooled = jnp.concatenate(rows, axis=0)

    logits = wdot(pooled, lw_ref[...]) + lb_ref[...]
    colm = jax.lax.broadcasted_iota(jnp.int32, (1, 128), 1) < 3
    z = jnp.where(colm, logits, -jnp.inf)
    mx = jnp.max(z, axis=1, keepdims=True)
    lse = mx + jnp.log(jnp.sum(jnp.where(colm, jnp.exp(z - mx), 0.0),
                               axis=1, keepdims=True))
    out_ref[...] = z - lse


def kernel(x, edge_index, batch_index, topk_weight,
           W0, b0, W1, b1, W2, b2,
           g0, be0, g1, be1, g2, be2, linW, linb):
    raw = (x * topk_weight[None, :]).sum(axis=-1)
    smax_g = jax.ops.segment_max(raw, batch_index, num_segments=_G)
    e = jnp.exp(raw - smax_g[batch_index])
    ssum = jax.ops.segment_sum(e, batch_index, num_segments=_G)
    score = e / ssum[batch_index]
    scmax = jax.ops.segment_max(score, batch_index,
                                num_segments=_G)[batch_index] - 1e-7
    mask = score > jnp.minimum(scmax, _MIN_SCORE)
    k = jnp.sum(mask.astype(jnp.int32))
    new_idx = jax.lax.associative_scan(jnp.add, mask.astype(jnp.int32)) - 1

    perm = jnp.nonzero(mask, size=_K_MAX, fill_value=_N)[0]
    sperm = jnp.take(score, perm, mode='fill', fill_value=0.0)
    xperm = jnp.take(x, perm, axis=0, mode='fill', fill_value=0.0)
    xs = xperm * sperm[:, None]
    bs = jnp.take(batch_index, perm, mode='fill',
                  fill_value=_G).astype(jnp.int32)

    row, col = edge_index[0], edge_index[1]
    evalid = mask[row] & mask[col]
    pos = jax.lax.associative_scan(jnp.add, evalid.astype(jnp.int32)) - 1
    posc = jnp.where(evalid, pos, _EK_MAX)
    er = jnp.minimum(new_idx[row], _K_MAX - 1)
    ec = jnp.minimum(new_idx[col], _K_MAX - 1)
    cer = jnp.zeros((_EK_MAX,), jnp.int32).at[posc].set(er, mode='drop')
    cec = jnp.zeros((_EK_MAX,), jnp.int32).at[posc].set(ec, mode='drop')
    ek = jnp.minimum(jnp.sum(evalid.astype(jnp.int32)), _EK_MAX)

    scal = jnp.stack([k, ek]).astype(jnp.int32)

    xs8 = jnp.pad(xs, ((0, 0), (0, 6)))
    W0p = jnp.pad(W0, ((0, 6), (0, 0)))
    lWp = jnp.pad(linW, ((0, 0), (0, 125)))
    lbp = jnp.pad(linb, (0, 125)).reshape(1, 128)
    r1 = lambda a: a.reshape(1, -1)

    smem = pl.BlockSpec(memory_space=pltpu.SMEM)
    vmem = pl.BlockSpec(memory_space=pltpu.VMEM)
    out = pl.pallas_call(
        _net_kernel,
        out_shape=jax.ShapeDtypeStruct((_G, 128), jnp.float32),
        in_specs=[smem, smem, smem] + [vmem] * 16,
        out_specs=vmem,
        scratch_shapes=[pltpu.VMEM((_K_MAX, _K_MAX), jnp.float32)],
    )(scal, cer, cec, bs.reshape(_K_MAX, 1), xs8,
      W0p, r1(b0), r1(g0), r1(be0),
      W1, r1(b1), r1(g1), r1(be1),
      W2, r1(b2), r1(g2), r1(be2),
      lWp, lbp)
    return out[:, :3]


# speedup vs baseline: 1.8234x; 1.0398x over previous
import jax
import jax.numpy as jnp
from jax.experimental import pallas as pl
from jax.experimental.pallas import tpu as pltpu

_N = 10000
_G = 64
_K_MAX = 1024
_EK_MAX = 8192
_MIN_SCORE = 0.1


def _net_kernel(scal_ref, cer_ref, cec_ref, bs_ref, xs_ref,
                w0_ref, b0_ref, g0_ref, be0_ref,
                w1_ref, b1_ref, g1_ref, be1_ref,
                w2_ref, b2_ref, g2_ref, be2_ref,
                lw_ref, lb_ref, out_ref, adj_ref):
    k = scal_ref[0]
    ek = scal_ref[1]
    kf = k.astype(jnp.float32)

    adj_ref[...] = jnp.zeros((_K_MAX, _K_MAX), jnp.float32)
    lane = jax.lax.broadcasted_iota(jnp.int32, (1, _K_MAX), 1)

    def body(i, carry):
        r = cer_ref[i]
        c = cec_ref[i]
        adj_ref[pl.ds(c, 1), :] += (lane == r).astype(jnp.float32)
        return carry

    jax.lax.fori_loop(0, ek, body, 0)
    cnt = adj_ref[...]

    riota = jax.lax.broadcasted_iota(jnp.int32, (_K_MAX, 1), 0)
    validf = (riota < k).astype(jnp.float32)
    deg = jnp.sum(cnt, axis=1, keepdims=True) + validf
    dinv = jnp.where(deg > 0, jax.lax.rsqrt(jnp.maximum(deg, 1e-12)), 0.0)

    def prop(h):
        hd = h * dinv
        out = jnp.dot(cnt, hd, preferred_element_type=jnp.float32, precision=jax.lax.Precision.HIGHEST)
        return out * dinv + (validf * dinv * dinv) * h

    def bn(h, g, be):
        m = jnp.sum(h * validf, axis=0, keepdims=True) / kf
        c = (h - m) * validf
        v = jnp.sum(c * c, axis=0, keepdims=True) / kf
        return (h - m) * jax.lax.rsqrt(v + 1e-5) * g + be

    def wdot(a, b):
        return jnp.dot(a.astype(jnp.bfloat16), b.astype(jnp.bfloat16),
                       preferred_element_type=jnp.float32)

    h = wdot(xs_ref[...], w0_ref[...])
    h = jax.nn.relu(prop(h) + b0_ref[...])
    h = bn(h, g0_ref[...], be0_ref[...])

    h = wdot(h, w1_ref[...])
    h = jax.nn.relu(prop(h) + b1_ref[...])
    h = bn(h, g1_ref[...], be1_ref[...])

    h = wdot(h, w2_ref[...])
    h = jax.nn.relu(prop(h) + b2_ref[...])
    h = bn(h, g2_ref[...], be2_ref[...])

    bsv = bs_ref[...]
    rows = []
    for g in range(_G):
        hm = jnp.where(bsv == g, h, -jnp.inf)
        rows.append(jnp.max(hm, axis=0, keepdims=True))
    pooled = jnp.concatenate(rows, axis=0)

    logits = wdot(pooled, lw_ref[...]) + lb_ref[...]
    colm = jax.lax.broadcasted_iota(jnp.int32, (1, 128), 1) < 3
    z = jnp.where(colm, logits, -jnp.inf)
    mx = jnp.max(z, axis=1, keepdims=True)
    lse = mx + jnp.log(jnp.sum(jnp.where(colm, jnp.exp(z - mx), 0.0),
                               axis=1, keepdims=True))
    out_ref[...] = z - lse


def kernel(x, edge_index, batch_index, topk_weight,
           W0, b0, W1, b1, W2, b2,
           g0, be0, g1, be1, g2, be2, linW, linb):
    raw = (x * topk_weight[None, :]).sum(axis=-1)
    smax_g = jax.ops.segment_max(raw, batch_index, num_segments=_G)
    e = jnp.exp(raw - smax_g[batch_index])
    ssum = jax.ops.segment_sum(e, batch_index, num_segments=_G)
    score = e / ssum[batch_index]
    scmax = jax.ops.segment_max(score, batch_index,
                                num_segments=_G)[batch_index] - 1e-7
    mask = score > jnp.minimum(scmax, _MIN_SCORE)
    k = jnp.sum(mask.astype(jnp.int32))
    new_idx = jax.lax.associative_scan(jnp.add, mask.astype(jnp.int32)) - 1

    perm = jnp.nonzero(mask, size=_K_MAX, fill_value=_N)[0]
    sperm = jnp.take(score, perm, mode='fill', fill_value=0.0)
    xperm = jnp.take(x, perm, axis=0, mode='fill', fill_value=0.0)
    xs = xperm * sperm[:, None]
    bs = jnp.take(batch_index, perm, mode='fill',
                  fill_value=_G).astype(jnp.int32)

    row, col = edge_index[0], edge_index[1]
    maskI = mask.astype(jnp.int32)
    evI = jnp.take(maskI, row) * jnp.take(maskI, col)
    evalid = evI == 1
    pos = jax.lax.associative_scan(jnp.add, evI) - 1
    posc = jnp.where(evalid, pos, _EK_MAX)
    er = jnp.minimum(jnp.take(new_idx, row), _K_MAX - 1)
    ec = jnp.minimum(jnp.take(new_idx, col), _K_MAX - 1)
    cer = jnp.zeros((_EK_MAX,), jnp.int32).at[posc].set(er, mode='drop')
    cec = jnp.zeros((_EK_MAX,), jnp.int32).at[posc].set(ec, mode='drop')
    ek = jnp.minimum(jnp.sum(evI), _EK_MAX)

    scal = jnp.stack([k, ek]).astype(jnp.int32)

    xs8 = jnp.pad(xs, ((0, 0), (0, 6)))
    W0p = jnp.pad(W0, ((0, 6), (0, 0)))
    lWp = jnp.pad(linW, ((0, 0), (0, 125)))
    lbp = jnp.pad(linb, (0, 125)).reshape(1, 128)
    r1 = lambda a: a.reshape(1, -1)

    smem = pl.BlockSpec(memory_space=pltpu.SMEM)
    vmem = pl.BlockSpec(memory_space=pltpu.VMEM)
    out = pl.pallas_call(
        _net_kernel,
        out_shape=jax.ShapeDtypeStruct((_G, 128), jnp.float32),
        in_specs=[smem, smem, smem] + [vmem] * 16,
        out_specs=vmem,
        scratch_shapes=[pltpu.VMEM((_K_MAX, _K_MAX), jnp.float32)],
    )(scal, cer, cec, bs.reshape(_K_MAX, 1), xs8,
      W0p, r1(b0), r1(g0), r1(be0),
      W1, r1(b1), r1(g1), r1(be1),
      W2, r1(b2), r1(g2), r1(be2),
      lWp, lbp)
    return out[:, :3]


# speedup vs baseline: 2.2896x; 1.2557x over previous
import jax
import jax.numpy as jnp
from jax.experimental import pallas as pl
from jax.experimental.pallas import tpu as pltpu

_N = 10000
_G = 64
_K_MAX = 1024
_MIN_SCORE = 0.1


def _net_kernel(scal_ref, bs_ref, adjc_ref, xs_ref,
                w0_ref, b0_ref, g0_ref, be0_ref,
                w1_ref, b1_ref, g1_ref, be1_ref,
                w2_ref, b2_ref, g2_ref, be2_ref,
                lw_ref, lb_ref, out_ref):
    k = scal_ref[0]
    kf = k.astype(jnp.float32)
    cnt = adjc_ref[...]

    riota = jax.lax.broadcasted_iota(jnp.int32, (_K_MAX, 1), 0)
    validf = (riota < k).astype(jnp.float32)
    deg = jnp.sum(cnt, axis=1, keepdims=True) + validf
    dinv = jnp.where(deg > 0, jax.lax.rsqrt(jnp.maximum(deg, 1e-12)), 0.0)

    def prop(h):
        hd = h * dinv
        out = jnp.dot(cnt, hd, preferred_element_type=jnp.float32,
                      precision=jax.lax.Precision.HIGHEST)
        return out * dinv + (validf * dinv * dinv) * h

    def bn(h, g, be):
        m = jnp.sum(h * validf, axis=0, keepdims=True) / kf
        c = (h - m) * validf
        v = jnp.sum(c * c, axis=0, keepdims=True) / kf
        return (h - m) * jax.lax.rsqrt(v + 1e-5) * g + be

    def wdot(a, b):
        return jnp.dot(a.astype(jnp.bfloat16), b.astype(jnp.bfloat16),
                       preferred_element_type=jnp.float32)

    h = wdot(xs_ref[...], w0_ref[...])
    h = jax.nn.relu(prop(h) + b0_ref[...])
    h = bn(h, g0_ref[...], be0_ref[...])

    h = wdot(h, w1_ref[...])
    h = jax.nn.relu(prop(h) + b1_ref[...])
    h = bn(h, g1_ref[...], be1_ref[...])

    h = wdot(h, w2_ref[...])
    h = jax.nn.relu(prop(h) + b2_ref[...])
    h = bn(h, g2_ref[...], be2_ref[...])

    bsv = bs_ref[...]
    rows = []
    for g in range(_G):
        hm = jnp.where(bsv == g, h, -jnp.inf)
        rows.append(jnp.max(hm, axis=0, keepdims=True))
    pooled = jnp.concatenate(rows, axis=0)

    logits = wdot(pooled, lw_ref[...]) + lb_ref[...]
    colm = jax.lax.broadcasted_iota(jnp.int32, (1, 128), 1) < 3
    z = jnp.where(colm, logits, -jnp.inf)
    mx = jnp.max(z, axis=1, keepdims=True)
    lse = mx + jnp.log(jnp.sum(jnp.where(colm, jnp.exp(z - mx), 0.0),
                               axis=1, keepdims=True))
    out_ref[...] = z - lse


def kernel(x, edge_index, batch_index, topk_weight,
           W0, b0, W1, b1, W2, b2,
           g0, be0, g1, be1, g2, be2, linW, linb):
    raw = (x * topk_weight[None, :]).sum(axis=-1)
    smax_g = jax.ops.segment_max(raw, batch_index, num_segments=_G)
    e = jnp.exp(raw - smax_g[batch_index])
    ssum = jax.ops.segment_sum(e, batch_index, num_segments=_G)
    score = e / ssum[batch_index]
    scmax = jax.ops.segment_max(score, batch_index,
                                num_segments=_G)[batch_index] - 1e-7
    mask = score > jnp.minimum(scmax, _MIN_SCORE)
    k = jnp.sum(mask.astype(jnp.int32))
    new_idx = jax.lax.associative_scan(jnp.add, mask.astype(jnp.int32)) - 1

    perm = jnp.nonzero(mask, size=_K_MAX, fill_value=_N)[0]
    sperm = jnp.take(score, perm, mode='fill', fill_value=0.0)
    xperm = jnp.take(x, perm, axis=0, mode='fill', fill_value=0.0)
    xs = xperm * sperm[:, None]
    bs = jnp.take(batch_index, perm, mode='fill',
                  fill_value=_G).astype(jnp.int32)

    row, col = edge_index[0], edge_index[1]
    maskI = mask.astype(jnp.int32)
    evI = jnp.take(maskI, row) * jnp.take(maskI, col)
    evalid = evI == 1
    er = jnp.minimum(jnp.take(new_idx, row), _K_MAX - 1)
    ec = jnp.minimum(jnp.take(new_idx, col), _K_MAX - 1)
    lin = jnp.where(evalid, ec * _K_MAX + er, _K_MAX * _K_MAX)
    adjc = jnp.zeros((_K_MAX * _K_MAX,), jnp.float32).at[lin].add(
        1.0, mode='drop').reshape(_K_MAX, _K_MAX)

    scal = k.reshape(1).astype(jnp.int32)

    xs8 = jnp.pad(xs, ((0, 0), (0, 6)))
    W0p = jnp.pad(W0, ((0, 6), (0, 0)))
    lWp = jnp.pad(linW, ((0, 0), (0, 125)))
    lbp = jnp.pad(linb, (0, 125)).reshape(1, 128)
    r1 = lambda a: a.reshape(1, -1)

    smem = pl.BlockSpec(memory_space=pltpu.SMEM)
    vmem = pl.BlockSpec(memory_space=pltpu.VMEM)
    out = pl.pallas_call(
        _net_kernel,
        out_shape=jax.ShapeDtypeStruct((_G, 128), jnp.float32),
        in_specs=[smem] + [vmem] * 17,
        out_specs=vmem,
    )(scal, bs.reshape(_K_MAX, 1), adjc, xs8,
      W0p, r1(b0), r1(g0), r1(be0),
      W1, r1(b1), r1(g1), r1(be1),
      W2, r1(b2), r1(g2), r1(be2),
      lWp, lbp)
    return out[:, :3]
